# baseline (speedup 1.0000x reference)
import functools

import jax
import jax.numpy as jnp
from jax.experimental import pallas as pl

T = 2048
D = 768
H = 12
KVH = 4
HD = 64
E = 8
K = 2
I = 1536
THETA = 10000.0
CLIP = 8.0
QW = H * HD
KVW = KVH * HD
HALF = HD // 2
SCALE = HD ** -0.5
REP = H // KVH

BT1 = 256
BTQ = 512
BK = 512
BTM = 512


def _silu(x):
    return x / (1.0 + jnp.exp(-x))


def _k1_body(hs_ref, w_ref, b_ref, wqkv_ref, cos_ref, sin_ref,
             q_ref, k_ref, v_ref):
    x = hs_ref[...]
    mu = jnp.mean(x, axis=-1, keepdims=True)
    var = jnp.mean((x - mu) ** 2, axis=-1, keepdims=True)
    xn = (x - mu) * jax.lax.rsqrt(var + 1e-5) * w_ref[...] + b_ref[...]
    qkv = jax.lax.dot_general(
        xn.astype(jnp.bfloat16), wqkv_ref[...].astype(jnp.bfloat16),
        (((1,), (1,)), ((), ())), preferred_element_type=jnp.float32)
    qkv = jnp.clip(qkv, -CLIP, CLIP)
    cos = cos_ref[...]
    sin = sin_ref[...]
    for h in range(H):
        base = h * HD
        x1 = qkv[:, base:base + HALF]
        x2 = qkv[:, base + HALF:base + HD]
        q_ref[h, :, :HALF] = ((x1 * cos - x2 * sin) * SCALE).astype(jnp.bfloat16)
        q_ref[h, :, HALF:] = ((x1 * sin + x2 * cos) * SCALE).astype(jnp.bfloat16)
    for h in range(KVH):
        base = QW + h * HD
        x1 = qkv[:, base:base + HALF]
        x2 = qkv[:, base + HALF:base + HD]
        k_ref[h, :, :HALF] = (x1 * cos - x2 * sin).astype(jnp.bfloat16)
        k_ref[h, :, HALF:] = (x1 * sin + x2 * cos).astype(jnp.bfloat16)
        vbase = QW + KVW + h * HD
        v_ref[h, :, :] = qkv[:, vbase:vbase + HD].astype(jnp.bfloat16)


def _run_k1(hs, n1w, n1b, wqkv, cos, sin):
    grid = (T // BT1,)
    return pl.pallas_call(
        _k1_body,
        grid=grid,
        in_specs=[
            pl.BlockSpec((BT1, D), lambda i: (i, 0)),
            pl.BlockSpec((D,), lambda i: (0,)),
            pl.BlockSpec((D,), lambda i: (0,)),
            pl.BlockSpec((QW + 2 * KVW, D), lambda i: (0, 0)),
            pl.BlockSpec((BT1, HALF), lambda i: (i, 0)),
            pl.BlockSpec((BT1, HALF), lambda i: (i, 0)),
        ],
        out_specs=[
            pl.BlockSpec((H, BT1, HD), lambda i: (0, i, 0)),
            pl.BlockSpec((KVH, BT1, HD), lambda i: (0, i, 0)),
            pl.BlockSpec((KVH, BT1, HD), lambda i: (0, i, 0)),
        ],
        out_shape=[
            jax.ShapeDtypeStruct((H, T, HD), jnp.bfloat16),
            jax.ShapeDtypeStruct((KVH, T, HD), jnp.bfloat16),
            jax.ShapeDtypeStruct((KVH, T, HD), jnp.bfloat16),
        ],
    )(hs, n1w, n1b, wqkv, cos, sin)


def _k2_body(q_ref, k_ref, v_ref, o_ref):
    qi = pl.program_id(1)
    q = q_ref[0]

    def step(j, carry, masked):
        m, l, acc = carry
        kc = k_ref[0, pl.ds(j * BK, BK), :]
        s = jax.lax.dot_general(q, kc, (((1,), (1,)), ((), ())),
                                preferred_element_type=jnp.float32)
        if masked:
            rows = jax.lax.broadcasted_iota(jnp.int32, (BTQ, BK), 0)
            cols = jax.lax.broadcasted_iota(jnp.int32, (BTQ, BK), 1)
            s = jnp.where(rows >= cols, s, -1e30)
        mc = jnp.max(s, axis=-1, keepdims=True)
        mn = jnp.maximum(m, mc)
        p = jnp.exp(s - mn)
        corr = jnp.exp(m - mn)
        l = l * corr + jnp.sum(p, axis=-1, keepdims=True)
        vc = v_ref[0, pl.ds(j * BK, BK), :]
        acc = acc * corr + jnp.dot(p.astype(jnp.bfloat16), vc,
                                   preferred_element_type=jnp.float32)
        return mn, l, acc

    m0 = jnp.full((BTQ, 1), -1e30, jnp.float32)
    l0 = jnp.zeros((BTQ, 1), jnp.float32)
    a0 = jnp.zeros((BTQ, HD), jnp.float32)
    carry = jax.lax.fori_loop(0, qi, lambda j, c: step(j, c, False),
                              (m0, l0, a0))
    m, l, acc = step(qi, carry, True)
    o_ref[0] = acc / l


def _run_k2(q, k, v):
    grid = (H, T // BTQ)
    return pl.pallas_call(
        _k2_body,
        grid=grid,
        in_specs=[
            pl.BlockSpec((1, BTQ, HD), lambda h, i: (h, i, 0)),
            pl.BlockSpec((1, T, HD), lambda h, i: (h // REP, 0, 0)),
            pl.BlockSpec((1, T, HD), lambda h, i: (h // REP, 0, 0)),
        ],
        out_specs=pl.BlockSpec((1, BTQ, HD), lambda h, i: (h, i, 0)),
        out_shape=jax.ShapeDtypeStruct((H, T, HD), jnp.float32),
    )(q, k, v)


def _k3_body(attn_ref, wout_ref, res_ref, w_ref, b_ref, wr_ref,
             h_ref, x2_ref, comb_ref):
    a = jnp.concatenate([attn_ref[h] for h in range(H)], axis=-1)
    a = a.astype(jnp.bfloat16)
    h = res_ref[...] + jax.lax.dot_general(
        a, wout_ref[...].astype(jnp.bfloat16), (((1,), (1,)), ((), ())),
        preferred_element_type=jnp.float32)
    h_ref[...] = h
    mu = jnp.mean(h, axis=-1, keepdims=True)
    var = jnp.mean((h - mu) ** 2, axis=-1, keepdims=True)
    x2 = (h - mu) * jax.lax.rsqrt(var + 1e-5) * w_ref[...] + b_ref[...]
    x2_ref[...] = x2
    logits = jax.lax.dot_general(x2, wr_ref[...], (((1,), (1,)), ((), ())),
                                 preferred_element_type=jnp.float32)
    mx = jnp.max(logits, axis=-1, keepdims=True)
    ex = jnp.exp(logits - mx)
    w_all = ex / jnp.sum(ex, axis=-1, keepdims=True)
    idx = jax.lax.broadcasted_iota(jnp.int32, (BT1, E), 1)
    m1 = jnp.max(w_all, axis=-1, keepdims=True)
    am1 = jnp.min(jnp.where(w_all == m1, idx, E), axis=-1, keepdims=True)
    is1 = idx == am1
    w_rest = jnp.where(is1, -1.0, w_all)
    m2 = jnp.max(w_rest, axis=-1, keepdims=True)
    am2 = jnp.min(jnp.where(w_rest == m2, idx, E), axis=-1, keepdims=True)
    tot = m1 + m2
    comb_ref[...] = (jnp.where(is1, m1, 0.0)
                     + jnp.where(idx == am2, m2, 0.0)) / tot


def _run_k3(attn, wout, res, n2w, n2b, wr):
    grid = (T // BT1,)
    return pl.pallas_call(
        _k3_body,
        grid=grid,
        in_specs=[
            pl.BlockSpec((H, BT1, HD), lambda i: (0, i, 0)),
            pl.BlockSpec((D, QW), lambda i: (0, 0)),
            pl.BlockSpec((BT1, D), lambda i: (i, 0)),
            pl.BlockSpec((D,), lambda i: (0,)),
            pl.BlockSpec((D,), lambda i: (0,)),
            pl.BlockSpec((E, D), lambda i: (0, 0)),
        ],
        out_specs=[
            pl.BlockSpec((BT1, D), lambda i: (i, 0)),
            pl.BlockSpec((BT1, D), lambda i: (i, 0)),
            pl.BlockSpec((BT1, E), lambda i: (i, 0)),
        ],
        out_shape=[
            jax.ShapeDtypeStruct((T, D), jnp.float32),
            jax.ShapeDtypeStruct((T, D), jnp.float32),
            jax.ShapeDtypeStruct((T, E), jnp.float32),
        ],
    )(attn, wout, res, n2w, n2b, wr)


def _k4_body(x2_ref, comb_ref, res_ref, ws_ref, w2_ref, out_ref):
    e = pl.program_id(0)
    w1 = ws_ref[0, :I, :]
    v1 = ws_ref[0, I:, :]
    w2 = w2_ref[0]
    eid = jax.lax.broadcasted_iota(jnp.int32, (E, 1), 0)
    onehot = (eid == e).astype(jnp.float32)
    for tc in range(T // BTM):
        sl = pl.ds(tc * BTM, BTM)
        x = x2_ref[sl, :].astype(jnp.bfloat16)
        g = jax.lax.dot_general(x, w1, (((1,), (1,)), ((), ())),
                                preferred_element_type=jnp.float32)
        u = jax.lax.dot_general(x, v1, (((1,), (1,)), ((), ())),
                                preferred_element_type=jnp.float32)
        act = (_silu(g) * u).astype(jnp.bfloat16)
        y = jax.lax.dot_general(act, w2, (((1,), (1,)), ((), ())),
                                preferred_element_type=jnp.float32)
        wcol = jnp.dot(comb_ref[sl, :], onehot,
                       preferred_element_type=jnp.float32)

        @pl.when(e == 0)
        def _():
            out_ref[sl, :] = res_ref[sl, :] + wcol * y

        @pl.when(e > 0)
        def _():
            out_ref[sl, :] += wcol * y


def _run_k4(x2, comb, res, ws, w2s):
    grid = (E,)
    return pl.pallas_call(
        _k4_body,
        grid=grid,
        in_specs=[
            pl.BlockSpec((T, D), lambda e: (0, 0)),
            pl.BlockSpec((T, E), lambda e: (0, 0)),
            pl.BlockSpec((T, D), lambda e: (0, 0)),
            pl.BlockSpec((1, 2 * I, D), lambda e: (e, 0, 0)),
            pl.BlockSpec((1, D, I), lambda e: (e, 0, 0)),
        ],
        out_specs=pl.BlockSpec((T, D), lambda e: (0, 0)),
        out_shape=jax.ShapeDtypeStruct((T, D), jnp.float32),
    )(x2, comb, res, ws.astype(jnp.bfloat16), w2s.astype(jnp.bfloat16))


def kernel(position_ids, hidden_states, norm1_w, norm1_b, norm2_w, norm2_b,
           Wqkv, Wout, Wrouter, ws, w2s):
    inv = 1.0 / (THETA ** (jnp.arange(HALF, dtype=jnp.float32) / HALF))
    ang = position_ids.astype(jnp.float32)[:, None] * inv[None, :]
    cos = jnp.cos(ang)
    sin = jnp.sin(ang)

    q, k, v = _run_k1(hidden_states, norm1_w, norm1_b, Wqkv, cos, sin)
    attn = _run_k2(q, k, v)
    h, x2, comb = _run_k3(attn, Wout, hidden_states, norm2_w, norm2_b, Wrouter)
    return _run_k4(x2, comb, h, ws, w2s)


# speedup vs baseline: 1.0968x; 1.0968x over previous
import functools

import jax
import jax.numpy as jnp
from jax import lax
from jax.experimental import pallas as pl
from jax.experimental.pallas import tpu as pltpu
from jax.experimental.pallas import tpu_sc as plsc

T = 2048
D = 768
H = 12
KVH = 4
HD = 64
E = 8
K = 2
I = 1536
THETA = 10000.0
CLIP = 8.0
QW = H * HD
KVW = KVH * HD
HALF = HD // 2
SCALE = HD ** -0.5
REP = H // KVH

BT1 = 256
BTQ = 512
BK = 512

BCH = 256
NPAD = 6144
NCH = NPAD // BCH
NC = 2
NS = 16
NW = NC * NS
TPW = T // NW


def _silu(x):
    return x / (1.0 + jnp.exp(-x))


def _k1_body(hs_ref, w_ref, b_ref, wqkv_ref, cos_ref, sin_ref,
             q_ref, k_ref, v_ref):
    x = hs_ref[...]
    mu = jnp.mean(x, axis=-1, keepdims=True)
    var = jnp.mean((x - mu) ** 2, axis=-1, keepdims=True)
    xn = (x - mu) * jax.lax.rsqrt(var + 1e-5) * w_ref[...] + b_ref[...]
    qkv = jax.lax.dot_general(
        xn.astype(jnp.bfloat16), wqkv_ref[...].astype(jnp.bfloat16),
        (((1,), (1,)), ((), ())), preferred_element_type=jnp.float32)
    qkv = jnp.clip(qkv, -CLIP, CLIP)
    cos = cos_ref[...]
    sin = sin_ref[...]
    for h in range(H):
        base = h * HD
        x1 = qkv[:, base:base + HALF]
        x2 = qkv[:, base + HALF:base + HD]
        q_ref[h, :, :HALF] = ((x1 * cos - x2 * sin) * SCALE).astype(jnp.bfloat16)
        q_ref[h, :, HALF:] = ((x1 * sin + x2 * cos) * SCALE).astype(jnp.bfloat16)
    for h in range(KVH):
        base = QW + h * HD
        x1 = qkv[:, base:base + HALF]
        x2 = qkv[:, base + HALF:base + HD]
        k_ref[h, :, :HALF] = (x1 * cos - x2 * sin).astype(jnp.bfloat16)
        k_ref[h, :, HALF:] = (x1 * sin + x2 * cos).astype(jnp.bfloat16)
        vbase = QW + KVW + h * HD
        v_ref[h, :, :] = qkv[:, vbase:vbase + HD].astype(jnp.bfloat16)


def _run_k1(hs, n1w, n1b, wqkv, cos, sin):
    grid = (T // BT1,)
    return pl.pallas_call(
        _k1_body,
        grid=grid,
        in_specs=[
            pl.BlockSpec((BT1, D), lambda i: (i, 0)),
            pl.BlockSpec((D,), lambda i: (0,)),
            pl.BlockSpec((D,), lambda i: (0,)),
            pl.BlockSpec((QW + 2 * KVW, D), lambda i: (0, 0)),
            pl.BlockSpec((BT1, HALF), lambda i: (i, 0)),
            pl.BlockSpec((BT1, HALF), lambda i: (i, 0)),
        ],
        out_specs=[
            pl.BlockSpec((H, BT1, HD), lambda i: (0, i, 0)),
            pl.BlockSpec((KVH, BT1, HD), lambda i: (0, i, 0)),
            pl.BlockSpec((KVH, BT1, HD), lambda i: (0, i, 0)),
        ],
        out_shape=[
            jax.ShapeDtypeStruct((H, T, HD), jnp.bfloat16),
            jax.ShapeDtypeStruct((KVH, T, HD), jnp.bfloat16),
            jax.ShapeDtypeStruct((KVH, T, HD), jnp.bfloat16),
        ],
    )(hs, n1w, n1b, wqkv, cos, sin)


def _k2_body(q_ref, k_ref, v_ref, o_ref):
    qi = pl.program_id(1)
    q = q_ref[0]

    def step(j, carry, masked):
        m, l, acc = carry
        kc = k_ref[0, pl.ds(j * BK, BK), :]
        s = jax.lax.dot_general(q, kc, (((1,), (1,)), ((), ())),
                                preferred_element_type=jnp.float32)
        if masked:
            rows = jax.lax.broadcasted_iota(jnp.int32, (BTQ, BK), 0)
            cols = jax.lax.broadcasted_iota(jnp.int32, (BTQ, BK), 1)
            s = jnp.where(rows >= cols, s, -1e30)
        mc = jnp.max(s, axis=-1, keepdims=True)
        mn = jnp.maximum(m, mc)
        p = jnp.exp(s - mn)
        corr = jnp.exp(m - mn)
        l = l * corr + jnp.sum(p, axis=-1, keepdims=True)
        vc = v_ref[0, pl.ds(j * BK, BK), :]
        acc = acc * corr + jnp.dot(p.astype(jnp.bfloat16), vc,
                                   preferred_element_type=jnp.float32)
        return mn, l, acc

    m0 = jnp.full((BTQ, 1), -1e30, jnp.float32)
    l0 = jnp.zeros((BTQ, 1), jnp.float32)
    a0 = jnp.zeros((BTQ, HD), jnp.float32)
    carry = jax.lax.fori_loop(0, qi, lambda j, c: step(j, c, False),
                              (m0, l0, a0))
    m, l, acc = step(qi, carry, True)
    o_ref[0] = acc / l


def _run_k2(q, k, v):
    grid = (H, T // BTQ)
    return pl.pallas_call(
        _k2_body,
        grid=grid,
        in_specs=[
            pl.BlockSpec((1, BTQ, HD), lambda h, i: (h, i, 0)),
            pl.BlockSpec((1, T, HD), lambda h, i: (h // REP, 0, 0)),
            pl.BlockSpec((1, T, HD), lambda h, i: (h // REP, 0, 0)),
        ],
        out_specs=pl.BlockSpec((1, BTQ, HD), lambda h, i: (h, i, 0)),
        out_shape=jax.ShapeDtypeStruct((H, T, HD), jnp.float32),
    )(q, k, v)


def _k3_body(attn_ref, wout_ref, res_ref, w_ref, b_ref, wr_ref,
             h_ref, x2_ref, e1_ref, e2_ref, r1_ref, r2_ref,
             w1_ref, w2_ref, cnt_ref, run_ref):
    i = pl.program_id(0)
    a = jnp.concatenate([attn_ref[h] for h in range(H)], axis=-1)
    a = a.astype(jnp.bfloat16)
    h = res_ref[...] + jax.lax.dot_general(
        a, wout_ref[...].astype(jnp.bfloat16), (((1,), (1,)), ((), ())),
        preferred_element_type=jnp.float32)
    h_ref[...] = h
    mu = jnp.mean(h, axis=-1, keepdims=True)
    var = jnp.mean((h - mu) ** 2, axis=-1, keepdims=True)
    x2 = (h - mu) * jax.lax.rsqrt(var + 1e-5) * w_ref[...] + b_ref[...]
    x2_ref[...] = x2
    logits = jax.lax.dot_general(x2, wr_ref[...], (((1,), (1,)), ((), ())),
                                 preferred_element_type=jnp.float32)
    mx = jnp.max(logits, axis=-1, keepdims=True)
    ex = jnp.exp(logits - mx)
    w_all = ex / jnp.sum(ex, axis=-1, keepdims=True)
    idx = jax.lax.broadcasted_iota(jnp.int32, (BT1, E), 1)
    m1 = jnp.max(w_all, axis=-1, keepdims=True)
    am1 = jnp.min(jnp.where(w_all == m1, idx, E), axis=-1, keepdims=True)
    is1 = idx == am1
    w_rest = jnp.where(is1, -1.0, w_all)
    m2 = jnp.max(w_rest, axis=-1, keepdims=True)
    am2 = jnp.min(jnp.where(w_rest == m2, idx, E), axis=-1, keepdims=True)
    is2 = idx == am2
    tot = m1 + m2
    w1_ref[...] = m1 / tot
    w2_ref[...] = m2 / tot
    e1_ref[...] = am1.astype(jnp.float32)
    e2_ref[...] = am2.astype(jnp.float32)

    mask1 = is1.astype(jnp.float32)
    mask2 = is2.astype(jnp.float32)
    m = mask1 + mask2

    @pl.when(i == 0)
    def _():
        run_ref[...] = jnp.zeros((1, E), jnp.float32)

    base = run_ref[...]
    rows = jax.lax.broadcasted_iota(jnp.int32, (BT1, BT1), 0)
    cols = jax.lax.broadcasted_iota(jnp.int32, (BT1, BT1), 1)
    strict_tril = (rows > cols).astype(jnp.float32)
    excl = jax.lax.dot_general(strict_tril, m, (((1,), (0,)), ((), ())),
                               preferred_element_type=jnp.float32) + base
    r1_ref[...] = jnp.sum(mask1 * excl, axis=-1, keepdims=True)
    r2_ref[...] = jnp.sum(mask2 * excl, axis=-1, keepdims=True)
    run_ref[...] = base + jnp.sum(m, axis=0, keepdims=True)
    cnt_ref[...] = run_ref[...]


def _run_k3(attn, wout, res, n2w, n2b, wr):
    grid = (T // BT1,)
    return pl.pallas_call(
        _k3_body,
        grid=grid,
        in_specs=[
            pl.BlockSpec((H, BT1, HD), lambda i: (0, i, 0)),
            pl.BlockSpec((D, QW), lambda i: (0, 0)),
            pl.BlockSpec((BT1, D), lambda i: (i, 0)),
            pl.BlockSpec((D,), lambda i: (0,)),
            pl.BlockSpec((D,), lambda i: (0,)),
            pl.BlockSpec((E, D), lambda i: (0, 0)),
        ],
        out_specs=[
            pl.BlockSpec((BT1, D), lambda i: (i, 0)),
            pl.BlockSpec((BT1, D), lambda i: (i, 0)),
            pl.BlockSpec((BT1, 1), lambda i: (i, 0)),
            pl.BlockSpec((BT1, 1), lambda i: (i, 0)),
            pl.BlockSpec((BT1, 1), lambda i: (i, 0)),
            pl.BlockSpec((BT1, 1), lambda i: (i, 0)),
            pl.BlockSpec((BT1, 1), lambda i: (i, 0)),
            pl.BlockSpec((BT1, 1), lambda i: (i, 0)),
            pl.BlockSpec((1, E), lambda i: (0, 0)),
        ],
        out_shape=[
            jax.ShapeDtypeStruct((T, D), jnp.float32),
            jax.ShapeDtypeStruct((T, D), jnp.float32),
            jax.ShapeDtypeStruct((T, 1), jnp.float32),
            jax.ShapeDtypeStruct((T, 1), jnp.float32),
            jax.ShapeDtypeStruct((T, 1), jnp.float32),
            jax.ShapeDtypeStruct((T, 1), jnp.float32),
            jax.ShapeDtypeStruct((T, 1), jnp.float32),
            jax.ShapeDtypeStruct((T, 1), jnp.float32),
            jax.ShapeDtypeStruct((1, E), jnp.float32),
        ],
        scratch_shapes=[pltpu.VMEM((1, E), jnp.float32)],
    )(attn, wout, res, n2w, n2b, wr)


@functools.cache
def _sc_kernels():
    mesh = plsc.VectorSubcoreMesh(core_axis_name="c", subcore_axis_name="s")

    @functools.partial(
        pl.kernel,
        mesh=mesh,
        out_type=jax.ShapeDtypeStruct((NPAD, D), jnp.float32),
        scratch_types=[
            pltpu.VMEM((2 * TPW // 16, 16), jnp.int32),
            pltpu.VMEM((16, D), jnp.float32),
            pltpu.SemaphoreType.DMA,
        ],
    )
    def sc_dispatch(x2_hbm, idx_hbm, buf_hbm, idx_v, rows_v, sem):
        wid = lax.axis_index("s") * NC + lax.axis_index("c")
        pltpu.sync_copy(idx_hbm.at[wid], idx_v)
        for j in range(TPW // 16):
            tok = wid * TPW + j * 16
            pltpu.sync_copy(x2_hbm.at[pl.ds(tok, 16)], rows_v)
            c1 = pltpu.async_copy(rows_v, buf_hbm.at[idx_v.at[j]], sem)
            c2 = pltpu.async_copy(rows_v, buf_hbm.at[idx_v.at[j + TPW // 16]],
                                  sem)
            c1.wait()
            c2.wait()

    @functools.partial(
        pl.kernel,
        mesh=mesh,
        out_type=jax.ShapeDtypeStruct((2 * T, D), jnp.float32),
        scratch_types=[
            pltpu.VMEM((2 * TPW // 16, 16), jnp.int32),
            pltpu.VMEM((16, D), jnp.float32),
            pltpu.SemaphoreType.DMA,
        ],
    )
    def sc_combine_gather(y_hbm, idx_hbm, out_hbm, idx_v, rows_v, sem):
        wid = lax.axis_index("s") * NC + lax.axis_index("c")
        pltpu.sync_copy(idx_hbm.at[wid], idx_v)
        for j in range(2 * TPW // 16):
            k = j // (TPW // 16)
            jj = j % (TPW // 16)
            base = k * T + wid * TPW + jj * 16
            pltpu.async_copy(y_hbm.at[idx_v.at[j]], rows_v, sem).wait()
            pltpu.sync_copy(rows_v, out_hbm.at[pl.ds(base, 16)])

    return sc_dispatch, sc_combine_gather


def _k5_body(cmap_ref, buf_ref, ws_ref, w2_ref, y_ref):
    x = buf_ref[...].astype(jnp.bfloat16)
    w1 = ws_ref[0, :I, :].astype(jnp.bfloat16)
    v1 = ws_ref[0, I:, :].astype(jnp.bfloat16)
    w2 = w2_ref[0].astype(jnp.bfloat16)
    g = jax.lax.dot_general(x, w1, (((1,), (1,)), ((), ())),
                            preferred_element_type=jnp.float32)
    u = jax.lax.dot_general(x, v1, (((1,), (1,)), ((), ())),
                            preferred_element_type=jnp.float32)
    act = (_silu(g) * u).astype(jnp.bfloat16)
    y_ref[...] = jax.lax.dot_general(act, w2, (((1,), (1,)), ((), ())),
                                     preferred_element_type=jnp.float32)


def _run_k5(cmap, buf, ws, w2s):
    grid_spec = pltpu.PrefetchScalarGridSpec(
        num_scalar_prefetch=1,
        grid=(NCH,),
        in_specs=[
            pl.BlockSpec((BCH, D), lambda c, m: (c, 0)),
            pl.BlockSpec((1, 2 * I, D), lambda c, m: (m[c], 0, 0)),
            pl.BlockSpec((1, D, I), lambda c, m: (m[c], 0, 0)),
        ],
        out_specs=pl.BlockSpec((BCH, D), lambda c, m: (c, 0)),
    )
    return pl.pallas_call(
        _k5_body,
        grid_spec=grid_spec,
        out_shape=jax.ShapeDtypeStruct((NPAD, D), jnp.float32),
    )(cmap, buf, ws, w2s)


def _k6_body(h_ref, y1_ref, y2_ref, w1_ref, w2_ref, out_ref):
    out_ref[...] = (h_ref[...] + w1_ref[...] * y1_ref[...]
                    + w2_ref[...] * y2_ref[...])


def _run_k6(h, y1, y2, w1n, w2n):
    grid = (T // BT1,)
    return pl.pallas_call(
        _k6_body,
        grid=grid,
        in_specs=[
            pl.BlockSpec((BT1, D), lambda i: (i, 0)),
            pl.BlockSpec((BT1, D), lambda i: (i, 0)),
            pl.BlockSpec((BT1, D), lambda i: (i, 0)),
            pl.BlockSpec((BT1, 1), lambda i: (i, 0)),
            pl.BlockSpec((BT1, 1), lambda i: (i, 0)),
        ],
        out_specs=pl.BlockSpec((BT1, D), lambda i: (i, 0)),
        out_shape=jax.ShapeDtypeStruct((T, D), jnp.float32),
    )(h, y1, y2, w1n, w2n)


def kernel(position_ids, hidden_states, norm1_w, norm1_b, norm2_w, norm2_b,
           Wqkv, Wout, Wrouter, ws, w2s):
    inv = 1.0 / (THETA ** (jnp.arange(HALF, dtype=jnp.float32) / HALF))
    ang = position_ids.astype(jnp.float32)[:, None] * inv[None, :]
    cos = jnp.cos(ang)
    sin = jnp.sin(ang)

    q, k, v = _run_k1(hidden_states, norm1_w, norm1_b, Wqkv, cos, sin)
    attn = _run_k2(q, k, v)
    (h, x2, e1f, e2f, r1f, r2f, w1n, w2n, cntf) = _run_k3(
        attn, Wout, hidden_states, norm2_w, norm2_b, Wrouter)

    cnt = cntf[0].astype(jnp.int32)
    padded = ((cnt + BCH - 1) // BCH) * BCH
    off = jnp.concatenate(
        [jnp.zeros((1,), jnp.int32), jnp.cumsum(padded)[:-1].astype(jnp.int32)])
    chunk_ids = jnp.arange(NCH, dtype=jnp.int32) * BCH
    cmap = (jnp.sum((chunk_ids[:, None] >= off[None, :]).astype(jnp.int32),
                    axis=1) - 1).astype(jnp.int32)

    e1 = e1f[:, 0].astype(jnp.int32)
    e2 = e2f[:, 0].astype(jnp.int32)
    slot1 = jnp.take(off, e1) + r1f[:, 0].astype(jnp.int32)
    slot2 = jnp.take(off, e2) + r2f[:, 0].astype(jnp.int32)
    scidx = jnp.concatenate(
        [slot1.reshape(NW, TPW // 16, 16), slot2.reshape(NW, TPW // 16, 16)],
        axis=1)

    sc_dispatch, sc_combine_gather = _sc_kernels()
    buf = sc_dispatch(x2, scidx)
    y = _run_k5(cmap, buf, ws, w2s)
    yg = sc_combine_gather(y, scidx)
    return _run_k6(h, yg[:T], yg[T:], w1n, w2n)


# speedup vs baseline: 1.1270x; 1.0276x over previous
import functools

import jax
import jax.numpy as jnp
from jax import lax
from jax.experimental import pallas as pl
from jax.experimental.pallas import tpu as pltpu
from jax.experimental.pallas import tpu_sc as plsc

T = 2048
D = 768
H = 12
KVH = 4
HD = 64
E = 8
K = 2
I = 1536
THETA = 10000.0
CLIP = 8.0
QW = H * HD
KVW = KVH * HD
HALF = HD // 2
SCALE = HD ** -0.5
REP = H // KVH

BT1 = 256
BTQ = 512
BK = 512

BCH = 256
NPAD = 6144
NCH = NPAD // BCH
NC = 2
NS = 16
NW = NC * NS
TPW = T // NW


def _silu(x):
    return x / (1.0 + jnp.exp(-x))


def _k1_body(hs_ref, w_ref, b_ref, wqkv_ref, cos_ref, sin_ref,
             q_ref, k_ref, v_ref):
    x = hs_ref[...]
    mu = jnp.mean(x, axis=-1, keepdims=True)
    var = jnp.mean((x - mu) ** 2, axis=-1, keepdims=True)
    xn = (x - mu) * jax.lax.rsqrt(var + 1e-5) * w_ref[...] + b_ref[...]
    qkv = jax.lax.dot_general(
        xn.astype(jnp.bfloat16), wqkv_ref[...].astype(jnp.bfloat16),
        (((1,), (1,)), ((), ())), preferred_element_type=jnp.float32)
    qkv = jnp.clip(qkv, -CLIP, CLIP)
    cos = cos_ref[...]
    sin = sin_ref[...]
    for h in range(H):
        base = h * HD
        x1 = qkv[:, base:base + HALF]
        x2 = qkv[:, base + HALF:base + HD]
        q_ref[h, :, :HALF] = ((x1 * cos - x2 * sin) * SCALE).astype(jnp.bfloat16)
        q_ref[h, :, HALF:] = ((x1 * sin + x2 * cos) * SCALE).astype(jnp.bfloat16)
    for h in range(KVH):
        base = QW + h * HD
        x1 = qkv[:, base:base + HALF]
        x2 = qkv[:, base + HALF:base + HD]
        k_ref[h, :, :HALF] = (x1 * cos - x2 * sin).astype(jnp.bfloat16)
        k_ref[h, :, HALF:] = (x1 * sin + x2 * cos).astype(jnp.bfloat16)
        vbase = QW + KVW + h * HD
        v_ref[h, :, :] = qkv[:, vbase:vbase + HD].astype(jnp.bfloat16)


def _run_k1(hs, n1w, n1b, wqkv, cos, sin):
    grid = (T // BT1,)
    return pl.pallas_call(
        _k1_body,
        grid=grid,
        in_specs=[
            pl.BlockSpec((BT1, D), lambda i: (i, 0)),
            pl.BlockSpec((D,), lambda i: (0,)),
            pl.BlockSpec((D,), lambda i: (0,)),
            pl.BlockSpec((QW + 2 * KVW, D), lambda i: (0, 0)),
            pl.BlockSpec((BT1, HALF), lambda i: (i, 0)),
            pl.BlockSpec((BT1, HALF), lambda i: (i, 0)),
        ],
        out_specs=[
            pl.BlockSpec((H, BT1, HD), lambda i: (0, i, 0)),
            pl.BlockSpec((KVH, BT1, HD), lambda i: (0, i, 0)),
            pl.BlockSpec((KVH, BT1, HD), lambda i: (0, i, 0)),
        ],
        out_shape=[
            jax.ShapeDtypeStruct((H, T, HD), jnp.bfloat16),
            jax.ShapeDtypeStruct((KVH, T, HD), jnp.bfloat16),
            jax.ShapeDtypeStruct((KVH, T, HD), jnp.bfloat16),
        ],
    )(hs, n1w, n1b, wqkv, cos, sin)


def _k2_body(q_ref, k_ref, v_ref, o_ref):
    qi = pl.program_id(1)
    q = q_ref[0]

    def step(j, carry, masked):
        m, l, acc = carry
        kc = k_ref[0, pl.ds(j * BK, BK), :]
        s = jax.lax.dot_general(q, kc, (((1,), (1,)), ((), ())),
                                preferred_element_type=jnp.float32)
        if masked:
            rows = jax.lax.broadcasted_iota(jnp.int32, (BTQ, BK), 0)
            cols = jax.lax.broadcasted_iota(jnp.int32, (BTQ, BK), 1)
            s = jnp.where(rows >= cols, s, -1e30)
        mc = jnp.max(s, axis=-1, keepdims=True)
        mn = jnp.maximum(m, mc)
        p = jnp.exp(s - mn)
        corr = jnp.exp(m - mn)
        l = l * corr + jnp.sum(p, axis=-1, keepdims=True)
        vc = v_ref[0, pl.ds(j * BK, BK), :]
        acc = acc * corr + jnp.dot(p.astype(jnp.bfloat16), vc,
                                   preferred_element_type=jnp.float32)
        return mn, l, acc

    m0 = jnp.full((BTQ, 1), -1e30, jnp.float32)
    l0 = jnp.zeros((BTQ, 1), jnp.float32)
    a0 = jnp.zeros((BTQ, HD), jnp.float32)
    carry = jax.lax.fori_loop(0, qi, lambda j, c: step(j, c, False),
                              (m0, l0, a0))
    m, l, acc = step(qi, carry, True)
    o_ref[0] = acc / l


def _run_k2(q, k, v):
    grid = (H, T // BTQ)
    return pl.pallas_call(
        _k2_body,
        grid=grid,
        in_specs=[
            pl.BlockSpec((1, BTQ, HD), lambda h, i: (h, i, 0)),
            pl.BlockSpec((1, T, HD), lambda h, i: (h // REP, 0, 0)),
            pl.BlockSpec((1, T, HD), lambda h, i: (h // REP, 0, 0)),
        ],
        out_specs=pl.BlockSpec((1, BTQ, HD), lambda h, i: (h, i, 0)),
        out_shape=jax.ShapeDtypeStruct((H, T, HD), jnp.float32),
    )(q, k, v)


def _k3_body(attn_ref, wout_ref, res_ref, w_ref, b_ref, wr_ref,
             h_ref, x2_ref, e1_ref, e2_ref, r1_ref, r2_ref,
             w1_ref, w2_ref, cnt_ref, run_ref):
    i = pl.program_id(0)
    a = jnp.concatenate([attn_ref[h] for h in range(H)], axis=-1)
    a = a.astype(jnp.bfloat16)
    h = res_ref[...] + jax.lax.dot_general(
        a, wout_ref[...].astype(jnp.bfloat16), (((1,), (1,)), ((), ())),
        preferred_element_type=jnp.float32)
    h_ref[...] = h
    mu = jnp.mean(h, axis=-1, keepdims=True)
    var = jnp.mean((h - mu) ** 2, axis=-1, keepdims=True)
    x2 = (h - mu) * jax.lax.rsqrt(var + 1e-5) * w_ref[...] + b_ref[...]
    x2_ref[...] = x2
    logits = jax.lax.dot_general(x2, wr_ref[...], (((1,), (1,)), ((), ())),
                                 preferred_element_type=jnp.float32)
    mx = jnp.max(logits, axis=-1, keepdims=True)
    ex = jnp.exp(logits - mx)
    w_all = ex / jnp.sum(ex, axis=-1, keepdims=True)
    idx = jax.lax.broadcasted_iota(jnp.int32, (BT1, E), 1)
    m1 = jnp.max(w_all, axis=-1, keepdims=True)
    am1 = jnp.min(jnp.where(w_all == m1, idx, E), axis=-1, keepdims=True)
    is1 = idx == am1
    w_rest = jnp.where(is1, -1.0, w_all)
    m2 = jnp.max(w_rest, axis=-1, keepdims=True)
    am2 = jnp.min(jnp.where(w_rest == m2, idx, E), axis=-1, keepdims=True)
    is2 = idx == am2
    tot = m1 + m2
    w1_ref[...] = m1 / tot
    w2_ref[...] = m2 / tot
    e1_ref[...] = am1.astype(jnp.float32)
    e2_ref[...] = am2.astype(jnp.float32)

    mask1 = is1.astype(jnp.float32)
    mask2 = is2.astype(jnp.float32)
    m = mask1 + mask2

    @pl.when(i == 0)
    def _():
        run_ref[...] = jnp.zeros((1, E), jnp.float32)

    base = run_ref[...]
    rows = jax.lax.broadcasted_iota(jnp.int32, (BT1, BT1), 0)
    cols = jax.lax.broadcasted_iota(jnp.int32, (BT1, BT1), 1)
    strict_tril = (rows > cols).astype(jnp.float32)
    excl = jax.lax.dot_general(strict_tril, m, (((1,), (0,)), ((), ())),
                               preferred_element_type=jnp.float32) + base
    r1_ref[...] = jnp.sum(mask1 * excl, axis=-1, keepdims=True)
    r2_ref[...] = jnp.sum(mask2 * excl, axis=-1, keepdims=True)
    run_ref[...] = base + jnp.sum(m, axis=0, keepdims=True)
    cnt_ref[...] = run_ref[...]


def _run_k3(attn, wout, res, n2w, n2b, wr):
    grid = (T // BT1,)
    return pl.pallas_call(
        _k3_body,
        grid=grid,
        in_specs=[
            pl.BlockSpec((H, BT1, HD), lambda i: (0, i, 0)),
            pl.BlockSpec((D, QW), lambda i: (0, 0)),
            pl.BlockSpec((BT1, D), lambda i: (i, 0)),
            pl.BlockSpec((D,), lambda i: (0,)),
            pl.BlockSpec((D,), lambda i: (0,)),
            pl.BlockSpec((E, D), lambda i: (0, 0)),
        ],
        out_specs=[
            pl.BlockSpec((BT1, D), lambda i: (i, 0)),
            pl.BlockSpec((BT1, D), lambda i: (i, 0)),
            pl.BlockSpec((BT1, 1), lambda i: (i, 0)),
            pl.BlockSpec((BT1, 1), lambda i: (i, 0)),
            pl.BlockSpec((BT1, 1), lambda i: (i, 0)),
            pl.BlockSpec((BT1, 1), lambda i: (i, 0)),
            pl.BlockSpec((BT1, 1), lambda i: (i, 0)),
            pl.BlockSpec((BT1, 1), lambda i: (i, 0)),
            pl.BlockSpec((1, E), lambda i: (0, 0)),
        ],
        out_shape=[
            jax.ShapeDtypeStruct((T, D), jnp.float32),
            jax.ShapeDtypeStruct((T, D), jnp.float32),
            jax.ShapeDtypeStruct((T, 1), jnp.float32),
            jax.ShapeDtypeStruct((T, 1), jnp.float32),
            jax.ShapeDtypeStruct((T, 1), jnp.float32),
            jax.ShapeDtypeStruct((T, 1), jnp.float32),
            jax.ShapeDtypeStruct((T, 1), jnp.float32),
            jax.ShapeDtypeStruct((T, 1), jnp.float32),
            jax.ShapeDtypeStruct((1, E), jnp.float32),
        ],
        scratch_shapes=[pltpu.VMEM((1, E), jnp.float32)],
    )(attn, wout, res, n2w, n2b, wr)


@functools.cache
def _sc_kernels():
    mesh = plsc.VectorSubcoreMesh(core_axis_name="c", subcore_axis_name="s")

    @functools.partial(
        pl.kernel,
        mesh=mesh,
        out_type=jax.ShapeDtypeStruct((NPAD, D), jnp.float32),
        scratch_types=[
            pltpu.VMEM((K, TPW), jnp.int32),
            pltpu.VMEM((TPW, D), jnp.float32),
            pltpu.SemaphoreType.DMA,
        ],
    )
    def sc_dispatch(x2_hbm, idx_hbm, buf_hbm, idx_v, rows_v, sem):
        wid = lax.axis_index("s") * NC + lax.axis_index("c")
        pltpu.sync_copy(idx_hbm.at[wid], idx_v)
        pltpu.sync_copy(x2_hbm.at[pl.ds(wid * TPW, TPW)], rows_v)
        c1 = pltpu.async_copy(rows_v, buf_hbm.at[idx_v.at[0]], sem)
        c2 = pltpu.async_copy(rows_v, buf_hbm.at[idx_v.at[1]], sem)
        c1.wait()
        c2.wait()

    @functools.partial(
        pl.kernel,
        mesh=mesh,
        out_type=jax.ShapeDtypeStruct((2 * T, D), jnp.float32),
        scratch_types=[
            pltpu.VMEM((K, TPW), jnp.int32),
            pltpu.VMEM((TPW, D), jnp.float32),
            pltpu.VMEM((TPW, D), jnp.float32),
            pltpu.SemaphoreType.DMA,
        ],
    )
    def sc_combine_gather(y_hbm, idx_hbm, out_hbm, idx_v, r1_v, r2_v, sem):
        wid = lax.axis_index("s") * NC + lax.axis_index("c")
        pltpu.sync_copy(idx_hbm.at[wid], idx_v)
        c1 = pltpu.async_copy(y_hbm.at[idx_v.at[0]], r1_v, sem)
        c2 = pltpu.async_copy(y_hbm.at[idx_v.at[1]], r2_v, sem)
        c1.wait()
        c2.wait()
        pltpu.sync_copy(r1_v, out_hbm.at[pl.ds(wid * TPW, TPW)])
        pltpu.sync_copy(r2_v, out_hbm.at[pl.ds(T + wid * TPW, TPW)])

    return sc_dispatch, sc_combine_gather


def _k5_body(cmap_ref, buf_ref, ws_ref, w2_ref, y_ref):
    x = buf_ref[...].astype(jnp.bfloat16)
    w1 = ws_ref[0, :I, :].astype(jnp.bfloat16)
    v1 = ws_ref[0, I:, :].astype(jnp.bfloat16)
    w2 = w2_ref[0].astype(jnp.bfloat16)
    g = jax.lax.dot_general(x, w1, (((1,), (1,)), ((), ())),
                            preferred_element_type=jnp.float32)
    u = jax.lax.dot_general(x, v1, (((1,), (1,)), ((), ())),
                            preferred_element_type=jnp.float32)
    act = (_silu(g) * u).astype(jnp.bfloat16)
    y_ref[...] = jax.lax.dot_general(act, w2, (((1,), (1,)), ((), ())),
                                     preferred_element_type=jnp.float32)


def _run_k5(cmap, buf, ws, w2s):
    grid_spec = pltpu.PrefetchScalarGridSpec(
        num_scalar_prefetch=1,
        grid=(NCH,),
        in_specs=[
            pl.BlockSpec((BCH, D), lambda c, m: (c, 0)),
            pl.BlockSpec((1, 2 * I, D), lambda c, m: (m[c], 0, 0)),
            pl.BlockSpec((1, D, I), lambda c, m: (m[c], 0, 0)),
        ],
        out_specs=pl.BlockSpec((BCH, D), lambda c, m: (c, 0)),
    )
    return pl.pallas_call(
        _k5_body,
        grid_spec=grid_spec,
        out_shape=jax.ShapeDtypeStruct((NPAD, D), jnp.float32),
    )(cmap, buf, ws, w2s)


def _k6_body(h_ref, y1_ref, y2_ref, w1_ref, w2_ref, out_ref):
    out_ref[...] = (h_ref[...] + w1_ref[...] * y1_ref[...]
                    + w2_ref[...] * y2_ref[...])


def _run_k6(h, y1, y2, w1n, w2n):
    grid = (T // BT1,)
    return pl.pallas_call(
        _k6_body,
        grid=grid,
        in_specs=[
            pl.BlockSpec((BT1, D), lambda i: (i, 0)),
            pl.BlockSpec((BT1, D), lambda i: (i, 0)),
            pl.BlockSpec((BT1, D), lambda i: (i, 0)),
            pl.BlockSpec((BT1, 1), lambda i: (i, 0)),
            pl.BlockSpec((BT1, 1), lambda i: (i, 0)),
        ],
        out_specs=pl.BlockSpec((BT1, D), lambda i: (i, 0)),
        out_shape=jax.ShapeDtypeStruct((T, D), jnp.float32),
    )(h, y1, y2, w1n, w2n)


def kernel(position_ids, hidden_states, norm1_w, norm1_b, norm2_w, norm2_b,
           Wqkv, Wout, Wrouter, ws, w2s):
    inv = 1.0 / (THETA ** (jnp.arange(HALF, dtype=jnp.float32) / HALF))
    ang = position_ids.astype(jnp.float32)[:, None] * inv[None, :]
    cos = jnp.cos(ang)
    sin = jnp.sin(ang)

    q, k, v = _run_k1(hidden_states, norm1_w, norm1_b, Wqkv, cos, sin)
    attn = _run_k2(q, k, v)
    (h, x2, e1f, e2f, r1f, r2f, w1n, w2n, cntf) = _run_k3(
        attn, Wout, hidden_states, norm2_w, norm2_b, Wrouter)

    cnt = cntf[0].astype(jnp.int32)
    padded = ((cnt + BCH - 1) // BCH) * BCH
    off = jnp.concatenate(
        [jnp.zeros((1,), jnp.int32), jnp.cumsum(padded)[:-1].astype(jnp.int32)])
    chunk_ids = jnp.arange(NCH, dtype=jnp.int32) * BCH
    cmap = (jnp.sum((chunk_ids[:, None] >= off[None, :]).astype(jnp.int32),
                    axis=1) - 1).astype(jnp.int32)

    e1 = e1f[:, 0].astype(jnp.int32)
    e2 = e2f[:, 0].astype(jnp.int32)
    slot1 = jnp.take(off, e1) + r1f[:, 0].astype(jnp.int32)
    slot2 = jnp.take(off, e2) + r2f[:, 0].astype(jnp.int32)
    scidx = jnp.stack(
        [slot1.reshape(NW, TPW), slot2.reshape(NW, TPW)], axis=1)

    sc_dispatch, sc_combine_gather = _sc_kernels()
    buf = sc_dispatch(x2, scidx)
    y = _run_k5(cmap, buf, ws, w2s)
    yg = sc_combine_gather(y, scidx)
    return _run_k6(h, yg[:T], yg[T:], w1n, w2n)


# speedup vs baseline: 1.1476x; 1.0182x over previous
import functools

import jax
import jax.numpy as jnp
from jax import lax
from jax.experimental import pallas as pl
from jax.experimental.pallas import tpu as pltpu
from jax.experimental.pallas import tpu_sc as plsc

T = 2048
D = 768
H = 12
KVH = 4
HD = 64
E = 8
K = 2
I = 1536
THETA = 10000.0
CLIP = 8.0
QW = H * HD
KVW = KVH * HD
HALF = HD // 2
SCALE = HD ** -0.5
REP = H // KVH

BT1 = 256
BTQ = 512
BK = 512

BCH = 256
NPAD = 6144
NCH = NPAD // BCH
NC = 2
NS = 16
NW = NC * NS
TPW = T // NW


def _silu(x):
    return x / (1.0 + jnp.exp(-x))


def _k1_body(hs_ref, w_ref, b_ref, wqkv_ref, cos_ref, sin_ref,
             q_ref, k_ref, v_ref):
    x = hs_ref[...]
    mu = jnp.mean(x, axis=-1, keepdims=True)
    var = jnp.mean((x - mu) ** 2, axis=-1, keepdims=True)
    xn = (x - mu) * jax.lax.rsqrt(var + 1e-5) * w_ref[...] + b_ref[...]
    qkv = jax.lax.dot_general(
        xn.astype(jnp.bfloat16), wqkv_ref[...].astype(jnp.bfloat16),
        (((1,), (1,)), ((), ())), preferred_element_type=jnp.float32)
    qkv = jnp.clip(qkv, -CLIP, CLIP)
    cos = cos_ref[...]
    sin = sin_ref[...]
    for h in range(H):
        base = h * HD
        x1 = qkv[:, base:base + HALF]
        x2 = qkv[:, base + HALF:base + HD]
        q_ref[h, :, :HALF] = ((x1 * cos - x2 * sin) * SCALE).astype(jnp.bfloat16)
        q_ref[h, :, HALF:] = ((x1 * sin + x2 * cos) * SCALE).astype(jnp.bfloat16)
    for h in range(KVH):
        base = QW + h * HD
        x1 = qkv[:, base:base + HALF]
        x2 = qkv[:, base + HALF:base + HD]
        k_ref[h, :, :HALF] = (x1 * cos - x2 * sin).astype(jnp.bfloat16)
        k_ref[h, :, HALF:] = (x1 * sin + x2 * cos).astype(jnp.bfloat16)
        vbase = QW + KVW + h * HD
        v_ref[h, :, :] = qkv[:, vbase:vbase + HD].astype(jnp.bfloat16)


def _run_k1(hs, n1w, n1b, wqkv, cos, sin):
    grid = (T // BT1,)
    return pl.pallas_call(
        _k1_body,
        grid=grid,
        in_specs=[
            pl.BlockSpec((BT1, D), lambda i: (i, 0)),
            pl.BlockSpec((D,), lambda i: (0,)),
            pl.BlockSpec((D,), lambda i: (0,)),
            pl.BlockSpec((QW + 2 * KVW, D), lambda i: (0, 0)),
            pl.BlockSpec((BT1, HALF), lambda i: (i, 0)),
            pl.BlockSpec((BT1, HALF), lambda i: (i, 0)),
        ],
        out_specs=[
            pl.BlockSpec((H, BT1, HD), lambda i: (0, i, 0)),
            pl.BlockSpec((KVH, BT1, HD), lambda i: (0, i, 0)),
            pl.BlockSpec((KVH, BT1, HD), lambda i: (0, i, 0)),
        ],
        out_shape=[
            jax.ShapeDtypeStruct((H, T, HD), jnp.bfloat16),
            jax.ShapeDtypeStruct((KVH, T, HD), jnp.bfloat16),
            jax.ShapeDtypeStruct((KVH, T, HD), jnp.bfloat16),
        ],
    )(hs, n1w, n1b, wqkv, cos, sin)


def _k2_body(q_ref, k_ref, v_ref, o_ref):
    qi = pl.program_id(1)
    q = q_ref[0]

    def step(j, carry, masked):
        m, l, acc = carry
        kc = k_ref[0, pl.ds(j * BK, BK), :]
        s = jax.lax.dot_general(q, kc, (((1,), (1,)), ((), ())),
                                preferred_element_type=jnp.float32)
        if masked:
            rows = jax.lax.broadcasted_iota(jnp.int32, (BTQ, BK), 0)
            cols = jax.lax.broadcasted_iota(jnp.int32, (BTQ, BK), 1)
            s = jnp.where(rows >= cols, s, -1e30)
        mc = jnp.max(s, axis=-1, keepdims=True)
        mn = jnp.maximum(m, mc)
        p = jnp.exp(s - mn)
        corr = jnp.exp(m - mn)
        l = l * corr + jnp.sum(p, axis=-1, keepdims=True)
        vc = v_ref[0, pl.ds(j * BK, BK), :]
        acc = acc * corr + jnp.dot(p.astype(jnp.bfloat16), vc,
                                   preferred_element_type=jnp.float32)
        return mn, l, acc

    m0 = jnp.full((BTQ, 1), -1e30, jnp.float32)
    l0 = jnp.zeros((BTQ, 1), jnp.float32)
    a0 = jnp.zeros((BTQ, HD), jnp.float32)
    carry = jax.lax.fori_loop(0, qi, lambda j, c: step(j, c, False),
                              (m0, l0, a0))
    m, l, acc = step(qi, carry, True)
    o_ref[0] = acc / l


def _run_k2(q, k, v):
    grid = (H, T // BTQ)
    return pl.pallas_call(
        _k2_body,
        grid=grid,
        in_specs=[
            pl.BlockSpec((1, BTQ, HD), lambda h, i: (h, i, 0)),
            pl.BlockSpec((1, T, HD), lambda h, i: (h // REP, 0, 0)),
            pl.BlockSpec((1, T, HD), lambda h, i: (h // REP, 0, 0)),
        ],
        out_specs=pl.BlockSpec((1, BTQ, HD), lambda h, i: (h, i, 0)),
        out_shape=jax.ShapeDtypeStruct((H, T, HD), jnp.float32),
    )(q, k, v)


def _k3_body(attn_ref, wout_ref, res_ref, w_ref, b_ref, wr_ref,
             h_ref, x2_ref, e1_ref, e2_ref, r1_ref, r2_ref,
             w1_ref, w2_ref, cnt_ref, run_ref):
    i = pl.program_id(0)
    a = jnp.concatenate([attn_ref[h] for h in range(H)], axis=-1)
    a = a.astype(jnp.bfloat16)
    h = res_ref[...] + jax.lax.dot_general(
        a, wout_ref[...].astype(jnp.bfloat16), (((1,), (1,)), ((), ())),
        preferred_element_type=jnp.float32)
    h_ref[...] = h
    mu = jnp.mean(h, axis=-1, keepdims=True)
    var = jnp.mean((h - mu) ** 2, axis=-1, keepdims=True)
    x2 = (h - mu) * jax.lax.rsqrt(var + 1e-5) * w_ref[...] + b_ref[...]
    x2_ref[...] = x2
    logits = jax.lax.dot_general(x2, wr_ref[...], (((1,), (1,)), ((), ())),
                                 preferred_element_type=jnp.float32)
    mx = jnp.max(logits, axis=-1, keepdims=True)
    ex = jnp.exp(logits - mx)
    w_all = ex / jnp.sum(ex, axis=-1, keepdims=True)
    idx = jax.lax.broadcasted_iota(jnp.int32, (BT1, E), 1)
    m1 = jnp.max(w_all, axis=-1, keepdims=True)
    am1 = jnp.min(jnp.where(w_all == m1, idx, E), axis=-1, keepdims=True)
    is1 = idx == am1
    w_rest = jnp.where(is1, -1.0, w_all)
    m2 = jnp.max(w_rest, axis=-1, keepdims=True)
    am2 = jnp.min(jnp.where(w_rest == m2, idx, E), axis=-1, keepdims=True)
    is2 = idx == am2
    tot = m1 + m2
    w1_ref[...] = m1 / tot
    w2_ref[...] = m2 / tot
    e1_ref[...] = am1.astype(jnp.float32)
    e2_ref[...] = am2.astype(jnp.float32)

    mask1 = is1.astype(jnp.float32)
    mask2 = is2.astype(jnp.float32)
    m = mask1 + mask2

    @pl.when(i == 0)
    def _():
        run_ref[...] = jnp.zeros((1, E), jnp.float32)

    base = run_ref[...]
    rows = jax.lax.broadcasted_iota(jnp.int32, (BT1, BT1), 0)
    cols = jax.lax.broadcasted_iota(jnp.int32, (BT1, BT1), 1)
    strict_tril = (rows > cols).astype(jnp.float32)
    excl = jax.lax.dot_general(strict_tril, m, (((1,), (0,)), ((), ())),
                               preferred_element_type=jnp.float32) + base
    r1_ref[...] = jnp.sum(mask1 * excl, axis=-1, keepdims=True)
    r2_ref[...] = jnp.sum(mask2 * excl, axis=-1, keepdims=True)
    run_ref[...] = base + jnp.sum(m, axis=0, keepdims=True)
    cnt_ref[...] = run_ref[...]


def _run_k3(attn, wout, res, n2w, n2b, wr):
    grid = (T // BT1,)
    return pl.pallas_call(
        _k3_body,
        grid=grid,
        in_specs=[
            pl.BlockSpec((H, BT1, HD), lambda i: (0, i, 0)),
            pl.BlockSpec((D, QW), lambda i: (0, 0)),
            pl.BlockSpec((BT1, D), lambda i: (i, 0)),
            pl.BlockSpec((D,), lambda i: (0,)),
            pl.BlockSpec((D,), lambda i: (0,)),
            pl.BlockSpec((E, D), lambda i: (0, 0)),
        ],
        out_specs=[
            pl.BlockSpec((BT1, D), lambda i: (i, 0)),
            pl.BlockSpec((BT1, D), lambda i: (i, 0)),
            pl.BlockSpec((BT1, 1), lambda i: (i, 0)),
            pl.BlockSpec((BT1, 1), lambda i: (i, 0)),
            pl.BlockSpec((BT1, 1), lambda i: (i, 0)),
            pl.BlockSpec((BT1, 1), lambda i: (i, 0)),
            pl.BlockSpec((BT1, 1), lambda i: (i, 0)),
            pl.BlockSpec((BT1, 1), lambda i: (i, 0)),
            pl.BlockSpec((1, E), lambda i: (0, 0)),
        ],
        out_shape=[
            jax.ShapeDtypeStruct((T, D), jnp.float32),
            jax.ShapeDtypeStruct((T, D), jnp.float32),
            jax.ShapeDtypeStruct((T, 1), jnp.float32),
            jax.ShapeDtypeStruct((T, 1), jnp.float32),
            jax.ShapeDtypeStruct((T, 1), jnp.float32),
            jax.ShapeDtypeStruct((T, 1), jnp.float32),
            jax.ShapeDtypeStruct((T, 1), jnp.float32),
            jax.ShapeDtypeStruct((T, 1), jnp.float32),
            jax.ShapeDtypeStruct((1, E), jnp.float32),
        ],
        scratch_shapes=[pltpu.VMEM((1, E), jnp.float32)],
    )(attn, wout, res, n2w, n2b, wr)


@functools.cache
def _sc_kernels():
    mesh = plsc.VectorSubcoreMesh(core_axis_name="c", subcore_axis_name="s")

    @functools.partial(
        pl.kernel,
        mesh=mesh,
        out_type=jax.ShapeDtypeStruct((NPAD, D), jnp.float32),
        scratch_types=[
            pltpu.VMEM((K, TPW), jnp.int32),
            pltpu.VMEM((TPW, D), jnp.float32),
            pltpu.SemaphoreType.DMA,
        ],
    )
    def sc_dispatch(x2_hbm, idx_hbm, buf_hbm, idx_v, rows_v, sem):
        wid = lax.axis_index("s") * NC + lax.axis_index("c")
        pltpu.sync_copy(idx_hbm.at[wid], idx_v)
        pltpu.sync_copy(x2_hbm.at[pl.ds(wid * TPW, TPW)], rows_v)
        c1 = pltpu.async_copy(rows_v, buf_hbm.at[idx_v.at[0]], sem)
        c2 = pltpu.async_copy(rows_v, buf_hbm.at[idx_v.at[1]], sem)
        c1.wait()
        c2.wait()

    @functools.partial(
        pl.kernel,
        mesh=mesh,
        out_type=jax.ShapeDtypeStruct((2 * T, D), jnp.float32),
        scratch_types=[
            pltpu.VMEM((K, TPW), jnp.int32),
            pltpu.VMEM((TPW, D), jnp.float32),
            pltpu.VMEM((TPW, D), jnp.float32),
            pltpu.SemaphoreType.DMA,
        ],
    )
    def sc_combine_gather(y_hbm, idx_hbm, out_hbm, idx_v, r1_v, r2_v, sem):
        wid = lax.axis_index("s") * NC + lax.axis_index("c")
        pltpu.sync_copy(idx_hbm.at[wid], idx_v)
        c1 = pltpu.async_copy(y_hbm.at[idx_v.at[0]], r1_v, sem)
        c2 = pltpu.async_copy(y_hbm.at[idx_v.at[1]], r2_v, sem)
        c1.wait()
        c2.wait()
        pltpu.sync_copy(r1_v, out_hbm.at[pl.ds(wid * TPW, TPW)])
        pltpu.sync_copy(r2_v, out_hbm.at[pl.ds(T + wid * TPW, TPW)])

    return sc_dispatch, sc_combine_gather


def _k5_body(cmap_ref, buf_ref, ws_ref, w2_ref, y_ref):
    c = pl.program_id(0)

    @pl.when(c < cmap_ref[NCH])
    def _():
        x = buf_ref[...].astype(jnp.bfloat16)
        w1 = ws_ref[0, :I, :].astype(jnp.bfloat16)
        v1 = ws_ref[0, I:, :].astype(jnp.bfloat16)
        w2 = w2_ref[0].astype(jnp.bfloat16)
        g = jax.lax.dot_general(x, w1, (((1,), (1,)), ((), ())),
                                preferred_element_type=jnp.float32)
        u = jax.lax.dot_general(x, v1, (((1,), (1,)), ((), ())),
                                preferred_element_type=jnp.float32)
        act = (_silu(g) * u).astype(jnp.bfloat16)
        y_ref[...] = jax.lax.dot_general(act, w2, (((1,), (1,)), ((), ())),
                                         preferred_element_type=jnp.float32)


def _run_k5(cmap, buf, ws, w2s):
    grid_spec = pltpu.PrefetchScalarGridSpec(
        num_scalar_prefetch=1,
        grid=(NCH,),
        in_specs=[
            pl.BlockSpec((BCH, D), lambda c, m: (c, 0)),
            pl.BlockSpec((1, 2 * I, D), lambda c, m: (m[c], 0, 0)),
            pl.BlockSpec((1, D, I), lambda c, m: (m[c], 0, 0)),
        ],
        out_specs=pl.BlockSpec((BCH, D), lambda c, m: (c, 0)),
    )
    return pl.pallas_call(
        _k5_body,
        grid_spec=grid_spec,
        out_shape=jax.ShapeDtypeStruct((NPAD, D), jnp.float32),
    )(cmap, buf, ws, w2s)


def _k6_body(h_ref, y1_ref, y2_ref, w1_ref, w2_ref, out_ref):
    out_ref[...] = (h_ref[...] + w1_ref[...] * y1_ref[...]
                    + w2_ref[...] * y2_ref[...])


def _run_k6(h, y1, y2, w1n, w2n):
    grid = (T // BT1,)
    return pl.pallas_call(
        _k6_body,
        grid=grid,
        in_specs=[
            pl.BlockSpec((BT1, D), lambda i: (i, 0)),
            pl.BlockSpec((BT1, D), lambda i: (i, 0)),
            pl.BlockSpec((BT1, D), lambda i: (i, 0)),
            pl.BlockSpec((BT1, 1), lambda i: (i, 0)),
            pl.BlockSpec((BT1, 1), lambda i: (i, 0)),
        ],
        out_specs=pl.BlockSpec((BT1, D), lambda i: (i, 0)),
        out_shape=jax.ShapeDtypeStruct((T, D), jnp.float32),
    )(h, y1, y2, w1n, w2n)


def kernel(position_ids, hidden_states, norm1_w, norm1_b, norm2_w, norm2_b,
           Wqkv, Wout, Wrouter, ws, w2s):
    inv = 1.0 / (THETA ** (jnp.arange(HALF, dtype=jnp.float32) / HALF))
    ang = position_ids.astype(jnp.float32)[:, None] * inv[None, :]
    cos = jnp.cos(ang)
    sin = jnp.sin(ang)

    q, k, v = _run_k1(hidden_states, norm1_w, norm1_b, Wqkv, cos, sin)
    attn = _run_k2(q, k, v)
    (h, x2, e1f, e2f, r1f, r2f, w1n, w2n, cntf) = _run_k3(
        attn, Wout, hidden_states, norm2_w, norm2_b, Wrouter)

    cnt = cntf[0].astype(jnp.int32)
    padded = ((cnt + BCH - 1) // BCH) * BCH
    off = jnp.concatenate(
        [jnp.zeros((1,), jnp.int32), jnp.cumsum(padded)[:-1].astype(jnp.int32)])
    chunk_ids = jnp.arange(NCH, dtype=jnp.int32) * BCH
    cmap = (jnp.sum((chunk_ids[:, None] >= off[None, :]).astype(jnp.int32),
                    axis=1) - 1).astype(jnp.int32)
    used = (jnp.sum(padded) // BCH).astype(jnp.int32)
    cmap = jnp.concatenate([cmap, used[None]])

    e1 = e1f[:, 0].astype(jnp.int32)
    e2 = e2f[:, 0].astype(jnp.int32)
    slot1 = jnp.take(off, e1) + r1f[:, 0].astype(jnp.int32)
    slot2 = jnp.take(off, e2) + r2f[:, 0].astype(jnp.int32)
    scidx = jnp.stack(
        [slot1.reshape(NW, TPW), slot2.reshape(NW, TPW)], axis=1)

    sc_dispatch, sc_combine_gather = _sc_kernels()
    buf = sc_dispatch(x2, scidx)
    y = _run_k5(cmap, buf, ws, w2s)
    yg = sc_combine_gather(y, scidx)
    return _run_k6(h, yg[:T], yg[T:], w1n, w2n)
